# Initial kernel scaffold; baseline (speedup 1.0000x reference)
import jax
import jax.numpy as jnp
from jax.experimental import pallas as pl


def kernel(x, edge_index, batch, g0, b0, W_feat, b_feat, W_conv0, b_conv0, W_conv1, b_conv1, W_conv2, b_conv2, W_edge, b_edge, W_node, b_node, g_ctx, b_ctx, W_ctx, c_ctx, g_obj, b_obj, W_obj, c_obj):
    raise NotImplementedError("write your pallas kernel here")


# speedup vs baseline: 14.9671x; 14.9671x over previous
import functools

import jax
import jax.numpy as jnp
from jax import lax
from jax.experimental import pallas as pl
from jax.experimental.pallas import tpu as pltpu
from jax.experimental.pallas import tpu_sc as plsc

N = 10000
E = 320000
F = 128
H = 128
C = 10
NG = 128

NC = 2
NS = 16
NW = NC * NS
CH = 100
NCHUNK = E // (NW * CH)
RPT = 624
RPT_LAST = N - (NS - 1) * RPT


def _deg_body(col3_hbm, ones_hbm, zeros1_hbm, out_hbm, colv, onesv, acc1):
    c = lax.axis_index("c")
    s = lax.axis_index("s")
    w = c * NS + s
    pltpu.sync_copy(col3_hbm.at[w], colv)
    pltpu.sync_copy(ones_hbm, onesv)

    @pl.when(s == 0)
    def _():
        pltpu.sync_copy(zeros1_hbm, acc1)

    plsc.subcore_barrier()

    def step(j, carry):
        pltpu.sync_copy(onesv.at[pl.ds(0, CH)], acc1.at[colv.at[j]], add=True)
        return carry

    lax.fori_loop(0, NCHUNK, step, 0)
    plsc.subcore_barrier()

    @pl.when(s == 0)
    def _():
        pltpu.sync_copy(acc1, out_hbm.at[c])


def _prop_body(hp_hbm, row3_hbm, col3_hbm, zeros_hbm, out_hbm,
               rowv, colv, buf, acc, sem):
    c = lax.axis_index("c")
    s = lax.axis_index("s")
    w = c * NS + s
    pltpu.sync_copy(row3_hbm.at[w], rowv)
    pltpu.sync_copy(col3_hbm.at[w], colv)
    r0 = s * RPT

    @pl.when(s < NS - 1)
    def _():
        pltpu.sync_copy(zeros_hbm.at[pl.ds(r0, RPT)], acc.at[pl.ds(r0, RPT)])

    @pl.when(s == NS - 1)
    def _():
        pltpu.sync_copy(zeros_hbm.at[pl.ds((NS - 1) * RPT, RPT_LAST)],
                        acc.at[pl.ds((NS - 1) * RPT, RPT_LAST)])

    plsc.subcore_barrier()

    def step(j, carry):
        pltpu.async_copy(hp_hbm.at[rowv.at[j]], buf, sem).wait()
        pltpu.sync_copy(buf, acc.at[colv.at[j]], add=True)
        return carry

    lax.fori_loop(0, NCHUNK, step, 0)
    plsc.subcore_barrier()

    @pl.when(s < NS - 1)
    def _():
        pltpu.sync_copy(acc.at[pl.ds(r0, RPT)],
                        out_hbm.at[c, pl.ds(r0, RPT)])

    @pl.when(s == NS - 1)
    def _():
        pltpu.sync_copy(acc.at[pl.ds((NS - 1) * RPT, RPT_LAST)],
                        out_hbm.at[c, pl.ds((NS - 1) * RPT, RPT_LAST)])


@functools.lru_cache(maxsize=None)
def _sc_kernels():
    mesh = plsc.VectorSubcoreMesh(
        core_axis_name="c", subcore_axis_name="s",
        num_cores=NC, num_subcores=NS)
    deg = pl.kernel(
        _deg_body,
        out_type=jax.ShapeDtypeStruct((NC, N), jnp.float32),
        mesh=mesh,
        scratch_types=[
            pltpu.VMEM((NCHUNK, CH), jnp.int32),
            pltpu.VMEM((128,), jnp.float32),
            pltpu.VMEM_SHARED((N,), jnp.float32),
        ],
    )
    prop = pl.kernel(
        _prop_body,
        out_type=jax.ShapeDtypeStruct((NC, N, H), jnp.float32),
        mesh=mesh,
        scratch_types=[
            pltpu.VMEM((NCHUNK, CH), jnp.int32),
            pltpu.VMEM((NCHUNK, CH), jnp.int32),
            pltpu.VMEM((CH, H), jnp.float32),
            pltpu.VMEM_SHARED((N, H), jnp.float32),
            pltpu.SemaphoreType.DMA,
        ],
    )
    return deg, prop


def _rsqrt(v):
    r = lax.rsqrt(v)
    return r * (1.5 - 0.5 * v * r * r)


def _bn_body(x_ref, g_ref, b_ref, w_ref, t_ref):
    x = x_ref[...]
    mu = jnp.mean(x, axis=0, keepdims=True)
    xc = x - mu
    var = jnp.mean(xc * xc, axis=0, keepdims=True)
    bn = xc * _rsqrt(var + 1e-5) * g_ref[...] + b_ref[...]
    t_ref[...] = jnp.dot(bn, w_ref[...], preferred_element_type=jnp.float32, precision=lax.Precision.HIGHEST)


def _hp_body(t_ref, degT_ref, dinv_ref, hp_ref):
    deg = degT_ref[:, 0:1] + degT_ref[:, 1:2] + 1.0
    dinv = _rsqrt(deg)
    dinv_ref[...] = dinv
    hp_ref[...] = t_ref[...] * dinv


def _combine_body(p0_ref, p1_ref, hp_ref, dinv_ref, b_ref, w_ref, hpn_ref):
    dinv = dinv_ref[...]
    pre = dinv * (p0_ref[...] + p1_ref[...] + hp_ref[...]) + b_ref[...]
    hin = jnp.maximum(pre, 0.0)
    t = jnp.dot(hin, w_ref[...], preferred_element_type=jnp.float32, precision=lax.Precision.HIGHEST)
    hpn_ref[...] = t * dinv


def _tail_body(p0_ref, p1_ref, hp_ref, dinv_ref, b_ref, batch_ref,
               wd_ref, zb_ref, gc_ref, bc_ref, wc_ref, cc_ref,
               go_ref, bo_ref, wo_ref, co_ref, outc_ref, outo_ref):
    dinv = dinv_ref[...]
    pre = dinv * (p0_ref[...] + p1_ref[...] + hp_ref[...]) + b_ref[...]
    h = jnp.maximum(pre, 0.0)
    z = jnp.dot(h, wd_ref[...], preferred_element_type=jnp.float32, precision=lax.Precision.HIGHEST) + zb_ref[...]
    a0 = 1.0 / (1.0 + jnp.exp(-z))
    xw = jnp.concatenate([a0 * h, (1.0 - a0) * h], axis=1)
    onehotT = (batch_ref[...] ==
               lax.broadcasted_iota(jnp.int32, (NG, 1), 0)).astype(jnp.float32)
    pooled = jnp.dot(onehotT, xw, preferred_element_type=jnp.float32, precision=lax.Precision.HIGHEST)
    cnt = jnp.sum(onehotT, axis=1, keepdims=True)
    mean = pooled / jnp.maximum(cnt, 1.0)

    def bnorm(v, g, b):
        mu = jnp.mean(v, axis=0, keepdims=True)
        vc = v - mu
        var = jnp.mean(vc * vc, axis=0, keepdims=True)
        return vc * _rsqrt(var + 1e-5) * g + b

    mc = mean[:, :H]
    mo = mean[:, H:]
    outc_ref[...] = (jnp.dot(bnorm(mc, gc_ref[...], bc_ref[...]), wc_ref[...],
                             preferred_element_type=jnp.float32, precision=lax.Precision.HIGHEST) + cc_ref[...])
    outo_ref[...] = (jnp.dot(bnorm(mo, go_ref[...], bo_ref[...]), wo_ref[...],
                             preferred_element_type=jnp.float32, precision=lax.Precision.HIGHEST) + co_ref[...])


def _tc_call(body, out_shapes):
    return pl.pallas_call(body, out_shape=out_shapes)


def kernel(x, edge_index, batch, g0, b0, W_feat, b_feat, W_conv0, b_conv0,
           W_conv1, b_conv1, W_conv2, b_conv2, W_edge, b_edge, W_node, b_node,
           g_ctx, b_ctx, W_ctx, c_ctx, g_obj, b_obj, W_obj, c_obj):
    row3 = edge_index[0].reshape(NW, NCHUNK, CH)
    col3 = edge_index[1].reshape(NW, NCHUNK, CH)
    ones128 = jnp.ones((128,), jnp.float32)
    zeros1 = jnp.zeros((N,), jnp.float32)
    zerosNH = jnp.zeros((N, H), jnp.float32)
    batch_row = batch.reshape(1, N)

    _deg_kernel, _prop_kernel = _sc_kernels()
    degp = _deg_kernel(col3, ones128, zeros1)
    degT = degp.T

    t0 = _tc_call(_bn_body, jax.ShapeDtypeStruct((N, H), jnp.float32))(
        x, g0.reshape(1, F), b0.reshape(1, F), W_feat)

    dinv, hp = _tc_call(
        _hp_body, (jax.ShapeDtypeStruct((N, 1), jnp.float32),
                   jax.ShapeDtypeStruct((N, H), jnp.float32)))(t0, degT)

    biases = (b_feat, b_conv0, b_conv1, b_conv2)
    weights_next = (W_conv0, W_conv1, W_conv2)
    for l in range(3):
        p = _prop_kernel(hp, row3, col3, zerosNH)
        hp = _tc_call(_combine_body,
                      jax.ShapeDtypeStruct((N, H), jnp.float32))(
            p[0], p[1], hp, dinv, biases[l].reshape(1, H), weights_next[l])

    p = _prop_kernel(hp, row3, col3, zerosNH)
    wd = (W_node[:, 0:1] - W_node[:, 1:2])
    zb = (b_node[0] - b_node[1]).reshape(1, 1)
    outc, outo = _tc_call(
        _tail_body, (jax.ShapeDtypeStruct((NG, C), jnp.float32),
                     jax.ShapeDtypeStruct((NG, C), jnp.float32)))(
        p[0], p[1], hp, dinv, biases[3].reshape(1, H), batch_row,
        wd, zb, g_ctx.reshape(1, H), b_ctx.reshape(1, H), W_ctx,
        c_ctx.reshape(1, C), g_obj.reshape(1, H), b_obj.reshape(1, H), W_obj,
        c_obj.reshape(1, C))
    return (outc, outo)


# speedup vs baseline: 21.9956x; 1.4696x over previous
import functools

import jax
import jax.numpy as jnp
from jax import lax
from jax.experimental import pallas as pl
from jax.experimental.pallas import tpu as pltpu
from jax.experimental.pallas import tpu_sc as plsc

N = 10000
E = 320000
F = 128
H = 128
C = 10
NG = 128

NC = 2
NS = 16
NW = NC * NS
CH = 100
NCHUNK = E // (NW * CH)
HCH = NCHUNK // 2
RPT = 624
RPT_LAST = N - (NS - 1) * RPT


def _deg_body(col3_hbm, ones_hbm, zeros1_hbm, out_hbm, colv, onesv, acc1):
    c = lax.axis_index("c")
    s = lax.axis_index("s")
    w = c * NS + s
    pltpu.sync_copy(col3_hbm.at[w], colv)
    pltpu.sync_copy(ones_hbm, onesv)

    @pl.when(s == 0)
    def _():
        pltpu.sync_copy(zeros1_hbm, acc1)

    plsc.subcore_barrier()

    def step(j, carry):
        pltpu.sync_copy(onesv.at[pl.ds(0, CH)], acc1.at[colv.at[j]], add=True)
        return carry

    lax.fori_loop(0, NCHUNK, step, 0)
    plsc.subcore_barrier()

    @pl.when(s == 0)
    def _():
        pltpu.sync_copy(acc1, out_hbm.at[c])


def _prop_body(hp_hbm, row3_hbm, col3_hbm, zeros_hbm, out_hbm,
               rowv, colv, buf0, buf1, acc, sem0, sem1):
    c = lax.axis_index("c")
    s = lax.axis_index("s")
    w = c * NS + s
    pltpu.sync_copy(row3_hbm.at[w], rowv)
    pltpu.sync_copy(col3_hbm.at[w], colv)
    r0 = s * RPT

    @pl.when(s < NS - 1)
    def _():
        pltpu.sync_copy(zeros_hbm.at[pl.ds(r0, RPT)], acc.at[pl.ds(r0, RPT)])

    @pl.when(s == NS - 1)
    def _():
        pltpu.sync_copy(zeros_hbm.at[pl.ds((NS - 1) * RPT, RPT_LAST)],
                        acc.at[pl.ds((NS - 1) * RPT, RPT_LAST)])

    plsc.subcore_barrier()

    bufs = (buf0, buf1)
    sems = (sem0, sem1)
    for ph in range(2):
        pltpu.sync_copy(row3_hbm.at[w * 2 + ph], rowv)
        pltpu.sync_copy(col3_hbm.at[w * 2 + ph], colv)
        pltpu.async_copy(hp_hbm.at[rowv.at[0]], buf0, sem0)

        def step2(i, carry):
            j = i * 2
            for b in range(2):
                jb = j + b
                nxt = bufs[(b + 1) % 2]
                nsem = sems[(b + 1) % 2]

                @pl.when(jb + 1 < HCH)
                def _():
                    pltpu.async_copy(hp_hbm.at[rowv.at[jb + 1]], nxt, nsem)

                pltpu.make_async_copy(hp_hbm.at[rowv.at[jb]], bufs[b],
                                      sems[b]).wait()
                pltpu.sync_copy(bufs[b], acc.at[colv.at[jb]], add=True)
            return carry

        lax.fori_loop(0, HCH // 2, step2, 0)
    plsc.subcore_barrier()

    @pl.when(s < NS - 1)
    def _():
        pltpu.sync_copy(acc.at[pl.ds(r0, RPT)],
                        out_hbm.at[c, pl.ds(r0, RPT)])

    @pl.when(s == NS - 1)
    def _():
        pltpu.sync_copy(acc.at[pl.ds((NS - 1) * RPT, RPT_LAST)],
                        out_hbm.at[c, pl.ds((NS - 1) * RPT, RPT_LAST)])


@functools.lru_cache(maxsize=None)
def _sc_kernels():
    mesh = plsc.VectorSubcoreMesh(
        core_axis_name="c", subcore_axis_name="s",
        num_cores=NC, num_subcores=NS)
    deg = pl.kernel(
        _deg_body,
        out_type=jax.ShapeDtypeStruct((NC, N), jnp.float32),
        mesh=mesh,
        scratch_types=[
            pltpu.VMEM((NCHUNK, CH), jnp.int32),
            pltpu.VMEM((128,), jnp.float32),
            pltpu.VMEM_SHARED((N,), jnp.float32),
        ],
    )
    prop = pl.kernel(
        _prop_body,
        out_type=jax.ShapeDtypeStruct((NC, N, H), jnp.float32),
        mesh=mesh,
        scratch_types=[
            pltpu.VMEM((HCH, CH), jnp.int32),
            pltpu.VMEM((HCH, CH), jnp.int32),
            pltpu.VMEM((CH, H), jnp.float32),
            pltpu.VMEM((CH, H), jnp.float32),
            pltpu.VMEM_SHARED((N, H), jnp.float32),
            pltpu.SemaphoreType.DMA,
            pltpu.SemaphoreType.DMA,
        ],
    )
    return deg, prop


def _rsqrt(v):
    r = lax.rsqrt(v)
    return r * (1.5 - 0.5 * v * r * r)


def _bn_body(x_ref, g_ref, b_ref, w_ref, t_ref):
    x = x_ref[...]
    mu = jnp.mean(x, axis=0, keepdims=True)
    xc = x - mu
    var = jnp.mean(xc * xc, axis=0, keepdims=True)
    bn = xc * _rsqrt(var + 1e-5) * g_ref[...] + b_ref[...]
    t_ref[...] = jnp.dot(bn, w_ref[...], preferred_element_type=jnp.float32, precision=lax.Precision.HIGHEST)


def _hp_body(t_ref, degT_ref, dinv_ref, hp_ref):
    deg = degT_ref[:, 0:1] + degT_ref[:, 1:2] + 1.0
    dinv = _rsqrt(deg)
    dinv_ref[...] = dinv
    hp_ref[...] = t_ref[...] * dinv


def _combine_body(p0_ref, p1_ref, hp_ref, dinv_ref, b_ref, w_ref, hpn_ref):
    dinv = dinv_ref[...]
    pre = dinv * (p0_ref[...] + p1_ref[...] + hp_ref[...]) + b_ref[...]
    hin = jnp.maximum(pre, 0.0)
    t = jnp.dot(hin, w_ref[...], preferred_element_type=jnp.float32, precision=lax.Precision.HIGHEST)
    hpn_ref[...] = t * dinv


def _tail_body(p0_ref, p1_ref, hp_ref, dinv_ref, b_ref, batch_ref,
               wd_ref, zb_ref, gc_ref, bc_ref, wc_ref, cc_ref,
               go_ref, bo_ref, wo_ref, co_ref, outc_ref, outo_ref):
    dinv = dinv_ref[...]
    pre = dinv * (p0_ref[...] + p1_ref[...] + hp_ref[...]) + b_ref[...]
    h = jnp.maximum(pre, 0.0)
    z = jnp.dot(h, wd_ref[...], preferred_element_type=jnp.float32, precision=lax.Precision.HIGHEST) + zb_ref[...]
    a0 = 1.0 / (1.0 + jnp.exp(-z))
    xw = jnp.concatenate([a0 * h, (1.0 - a0) * h], axis=1)
    onehotT = (batch_ref[...] ==
               lax.broadcasted_iota(jnp.int32, (NG, 1), 0)).astype(jnp.float32)
    pooled = jnp.dot(onehotT, xw, preferred_element_type=jnp.float32, precision=lax.Precision.HIGHEST)
    cnt = jnp.sum(onehotT, axis=1, keepdims=True)
    mean = pooled / jnp.maximum(cnt, 1.0)

    def bnorm(v, g, b):
        mu = jnp.mean(v, axis=0, keepdims=True)
        vc = v - mu
        var = jnp.mean(vc * vc, axis=0, keepdims=True)
        return vc * _rsqrt(var + 1e-5) * g + b

    mc = mean[:, :H]
    mo = mean[:, H:]
    outc_ref[...] = (jnp.dot(bnorm(mc, gc_ref[...], bc_ref[...]), wc_ref[...],
                             preferred_element_type=jnp.float32, precision=lax.Precision.HIGHEST) + cc_ref[...])
    outo_ref[...] = (jnp.dot(bnorm(mo, go_ref[...], bo_ref[...]), wo_ref[...],
                             preferred_element_type=jnp.float32, precision=lax.Precision.HIGHEST) + co_ref[...])


def _tc_call(body, out_shapes):
    return pl.pallas_call(body, out_shape=out_shapes)


def kernel(x, edge_index, batch, g0, b0, W_feat, b_feat, W_conv0, b_conv0,
           W_conv1, b_conv1, W_conv2, b_conv2, W_edge, b_edge, W_node, b_node,
           g_ctx, b_ctx, W_ctx, c_ctx, g_obj, b_obj, W_obj, c_obj):
    row4 = edge_index[0].reshape(NW * 2, HCH, CH)
    col4 = edge_index[1].reshape(NW * 2, HCH, CH)
    col3 = edge_index[1].reshape(NW, NCHUNK, CH)
    ones128 = jnp.ones((128,), jnp.float32)
    zeros1 = jnp.zeros((N,), jnp.float32)
    zerosNH = jnp.zeros((N, H), jnp.float32)
    batch_row = batch.reshape(1, N)

    _deg_kernel, _prop_kernel = _sc_kernels()
    degp = _deg_kernel(col3, ones128, zeros1)
    degT = degp.T

    t0 = _tc_call(_bn_body, jax.ShapeDtypeStruct((N, H), jnp.float32))(
        x, g0.reshape(1, F), b0.reshape(1, F), W_feat)

    dinv, hp = _tc_call(
        _hp_body, (jax.ShapeDtypeStruct((N, 1), jnp.float32),
                   jax.ShapeDtypeStruct((N, H), jnp.float32)))(t0, degT)

    biases = (b_feat, b_conv0, b_conv1, b_conv2)
    weights_next = (W_conv0, W_conv1, W_conv2)
    for l in range(3):
        p = _prop_kernel(hp, row4, col4, zerosNH)
        hp = _tc_call(_combine_body,
                      jax.ShapeDtypeStruct((N, H), jnp.float32))(
            p[0], p[1], hp, dinv, biases[l].reshape(1, H), weights_next[l])

    p = _prop_kernel(hp, row4, col4, zerosNH)
    wd = (W_node[:, 0:1] - W_node[:, 1:2])
    zb = (b_node[0] - b_node[1]).reshape(1, 1)
    outc, outo = _tc_call(
        _tail_body, (jax.ShapeDtypeStruct((NG, C), jnp.float32),
                     jax.ShapeDtypeStruct((NG, C), jnp.float32)))(
        p[0], p[1], hp, dinv, biases[3].reshape(1, H), batch_row,
        wd, zb, g_ctx.reshape(1, H), b_ctx.reshape(1, H), W_ctx,
        c_ctx.reshape(1, C), g_obj.reshape(1, H), b_obj.reshape(1, H), W_obj,
        c_obj.reshape(1, C))
    return (outc, outo)
